# baseline (speedup 1.0000x reference)
import functools
import math

import jax
import jax.numpy as jnp
from jax import lax
from jax.experimental import pallas as pl
from jax.experimental.pallas import tpu as pltpu
from jax.experimental.pallas import tpu_sc as plsc

N = 10000
E = 320000
D = 128
DE = 16
NC, NS, L = 2, 16, 16
NW = NC * NS
CH = 80
NG = CH // L
EPT = E // NW
NCH1 = EPT // CH
EPS = E // NS
NCH2 = EPS // CH
DH = D // NC
NPAD = 10240
RPT = NPAD // NS
INV_SQRT_C = 1.0 / math.sqrt(float(D))
F32 = jnp.float32
I32 = jnp.int32

_mesh = plsc.VectorSubcoreMesh(
    core_axis_name="c", subcore_axis_name="s", num_cores=NC, num_subcores=NS)
_params = pltpu.CompilerParams(
    needs_layout_passes=False, use_tc_tiling_on_sc=False)


def _qkv_body(x_ref, wq_ref, bq_ref, wk_ref, bk_ref, wv_ref, bv_ref, w2p_ref,
              q_ref, k_ref, v_ref, q2_ref):
    xb = x_ref[...]
    q = jnp.dot(xb, wq_ref[...], preferred_element_type=F32) + bq_ref[...]
    k = jnp.dot(xb, wk_ref[...], preferred_element_type=F32) + bk_ref[...]
    v = jnp.dot(xb, wv_ref[...], preferred_element_type=F32) + bv_ref[...]
    q_ref[...] = q
    k_ref[...] = k
    v_ref[...] = v
    q2_ref[...] = jnp.dot(q, w2p_ref[...], preferred_element_type=F32)


def _qkv(x, Wq, bq, Wk, bk, Wv, bv, W2p):
    bm = 1000
    grid = (N // bm,)
    wspec = pl.BlockSpec((D, D), lambda i: (0, 0))
    bspec = pl.BlockSpec((1, D), lambda i: (0, 0))
    rspec = pl.BlockSpec((bm, D), lambda i: (i, 0))
    return pl.pallas_call(
        _qkv_body,
        grid=grid,
        in_specs=[rspec, wspec, bspec, wspec, bspec, wspec, bspec,
                  pl.BlockSpec((D, 32), lambda i: (0, 0))],
        out_specs=[rspec, rspec, rspec,
                   pl.BlockSpec((bm, 32), lambda i: (i, 0))],
        out_shape=[jax.ShapeDtypeStruct((N, D), F32),
                   jax.ShapeDtypeStruct((N, D), F32),
                   jax.ShapeDtypeStruct((N, D), F32),
                   jax.ShapeDtypeStruct((N, 32), F32)],
    )(x, Wq, bq, Wk, bk, Wv, bv, W2p)


@functools.partial(
    pl.kernel,
    out_type=(jax.ShapeDtypeStruct((E,), F32),
              jax.ShapeDtypeStruct((E,), F32),
              jax.ShapeDtypeStruct((NC, NPAD), F32),
              jax.ShapeDtypeStruct((NC, NPAD), F32)),
    mesh=_mesh,
    compiler_params=_params,
    scratch_types=[
        pltpu.VMEM((NCH1, CH), I32),
        pltpu.VMEM((NCH1, CH), I32),
        pltpu.VMEM((CH, D), F32),
        pltpu.VMEM((CH, D), F32),
        pltpu.VMEM((CH, D), F32),
        pltpu.VMEM((CH, D), F32),
        pltpu.VMEM((CH, 32), F32),
        pltpu.VMEM((CH, 32), F32),
        pltpu.VMEM((CH, DE), F32),
        pltpu.VMEM((CH, DE), F32),
        pltpu.VMEM((CH,), F32),
        pltpu.VMEM((CH,), F32),
        pltpu.VMEM((CH,), F32),
        pltpu.VMEM((CH,), F32),
        pltpu.VMEM((NPAD,), F32),
        pltpu.VMEM((NPAD,), F32),
        pltpu.VMEM((NS, RPT), F32),
        pltpu.VMEM((RPT,), F32),
        pltpu.VMEM_SHARED((NS, NPAD), F32),
        pltpu.VMEM_SHARED((NS, NPAD), F32),
        pltpu.SemaphoreType.DMA,
        pltpu.SemaphoreType.DMA,
        pltpu.SemaphoreType.DMA,
        pltpu.SemaphoreType.DMA,
    ])
def _pass1(qh, kh, q2h, eah, srch2, dsth2,
           e1h, e2h, s1ph, s2ph,
           dst_l, src_l, Qg0, Qg1, Kg0, Kg1, q2g0, q2g1, eag0, eag1,
           e1b0, e1b1, e2b0, e2b1, s1l, s2l, red, rowout, sh1, sh2,
           semG0, semG1, semW0, semW1):
    cid = lax.axis_index("c")
    sid = lax.axis_index("s")
    wid = cid * NS + sid
    zero = jnp.zeros((L,), F32)
    iot = lax.iota(I32, L)
    BUFS = ((Qg0, Kg0, q2g0, eag0, e1b0, e2b0, semG0, semW0),
            (Qg1, Kg1, q2g1, eag1, e1b1, e2b1, semG1, semW1))

    def zbody(i, _):
        s1l[pl.ds(i * L, L)] = zero
        s2l[pl.ds(i * L, L)] = zero
        return 0
    lax.fori_loop(0, NPAD // L, zbody, 0)

    pltpu.sync_copy(dsth2.at[pl.ds(wid * NCH1, NCH1), :], dst_l)
    pltpu.sync_copy(srch2.at[pl.ds(wid * NCH1, NCH1), :], src_l)

    def fire_inputs(c, b):
        Qg, Kg, q2g, eag, _, _, semG, _ = BUFS[b]
        off = (wid * NCH1 + c) * CH
        pltpu.async_copy(qh.at[dst_l.at[c]], Qg, semG)
        pltpu.async_copy(kh.at[src_l.at[c]], Kg, semG)
        pltpu.async_copy(q2h.at[dst_l.at[c]], q2g, semG)
        pltpu.async_copy(eah.at[pl.ds(off, CH), :], eag, semG)

    def wait_inputs(b):
        Qg, Kg, q2g, eag, _, _, semG, _ = BUFS[b]
        pltpu.make_async_copy(qh.at[pl.ds(0, CH), :], Qg, semG).wait()
        pltpu.make_async_copy(kh.at[pl.ds(0, CH), :], Kg, semG).wait()
        pltpu.make_async_copy(q2h.at[pl.ds(0, CH), :], q2g, semG).wait()
        pltpu.make_async_copy(eah.at[pl.ds(0, CH), :], eag, semG).wait()

    def fire_writes(c, b):
        _, _, _, _, e1b, e2b, _, semW = BUFS[b]
        off = (wid * NCH1 + c) * CH
        pltpu.async_copy(e1b, e1h.at[pl.ds(off, CH)], semW)
        pltpu.async_copy(e2b, e2h.at[pl.ds(off, CH)], semW)

    def drain_writes(b):
        _, _, _, _, e1b, e2b, _, semW = BUFS[b]
        pltpu.make_async_copy(e1h.at[pl.ds(0, CH)], e1b, semW).wait()
        pltpu.make_async_copy(e2h.at[pl.ds(0, CH)], e2b, semW).wait()

    rows_g = tuple(iot + g * L for g in range(NG))

    def compute(c, b):
        Qg, Kg, q2g, eag, e1b, e2b, _, _ = BUFS[b]

        def dotstep(jj, accs):
            out = list(accs)
            for u in range(4):
                jv = jnp.full((L,), 0, I32) + (jj * 4 + u)
                for g in range(NG):
                    q = plsc.load_gather(Qg, [rows_g[g], jv])
                    k = plsc.load_gather(Kg, [rows_g[g], jv])
                    out[g * 4 + u] = out[g * 4 + u] + q * k
            return tuple(out)
        accs = lax.fori_loop(0, D // 4, dotstep, (zero,) * (NG * 4))

        a2s = [zero] * NG
        for j in range(DE):
            jv = jnp.full((L,), j, I32)
            for g in range(NG):
                a2s[g] = a2s[g] + (plsc.load_gather(q2g, [rows_g[g], jv])
                                   * plsc.load_gather(eag, [rows_g[g], jv]))
        jvq = jnp.full((L,), DE, I32)
        for g in range(NG):
            a1 = ((accs[g * 4] + accs[g * 4 + 1])
                  + (accs[g * 4 + 2] + accs[g * 4 + 3])) * INV_SQRT_C
            qb = plsc.load_gather(q2g, [rows_g[g], jvq])
            a2 = (a2s[g] + qb) * INV_SQRT_C
            e1 = jnp.exp(a1)
            e2 = jnp.exp(a2)
            e1b[pl.ds(g * L, L)] = e1
            e2b[pl.ds(g * L, L)] = e2
            dstv = dst_l[c, pl.ds(g * L, L)]

    fire_inputs(0, 0)
    fire_inputs(1, 1)

    def pair(k, _):
        for b in (0, 1):
            c = k * 2 + b
            wait_inputs(b)

            @pl.when(k > 0)
            def _():
                drain_writes(b)
            compute(c, b)
            fire_writes(c, b)
            cn = c + 2

            @pl.when(cn < NCH1)
            def _():
                fire_inputs(cn, b)
        return 0
    lax.fori_loop(0, NCH1 // 2, pair, 0)
    wait_inputs(0)
    drain_writes(0)
    compute(NCH1 - 1, 0)
    fire_writes(NCH1 - 1, 0)
    drain_writes(1)
    drain_writes(0)

    pltpu.sync_copy(s1l, sh1.at[sid])
    pltpu.sync_copy(s2l, sh2.at[sid])
    plsc.subcore_barrier()
    for sh, outref in ((sh1, s1ph), (sh2, s2ph)):
        pltpu.sync_copy(sh.at[:, pl.ds(sid * RPT, RPT)], red)

        def rbody(g, _):
            acc = red[0, pl.ds(g * L, L)]
            for r in range(1, NS):
                acc = acc + red[r, pl.ds(g * L, L)]
            rowout[pl.ds(g * L, L)] = acc
            return 0
        lax.fori_loop(0, RPT // L, rbody, 0)
        pltpu.sync_copy(rowout, outref.at[cid, pl.ds(sid * RPT, RPT)])


NSEG = 5
NCH2S = NCH2 // NSEG


@functools.partial(
    pl.kernel,
    out_type=(jax.ShapeDtypeStruct((NC, NPAD, DH), F32),
              jax.ShapeDtypeStruct((NC, NPAD, 32), F32)),
    mesh=_mesh,
    compiler_params=_params,
    scratch_types=[
        pltpu.VMEM((NCH2S, CH), I32),
        pltpu.VMEM((NCH2S, CH), I32),
        pltpu.VMEM((CH, DH), F32),
        pltpu.VMEM((CH, DH), F32),
        pltpu.VMEM((CH, DH), F32),
        pltpu.VMEM((CH, DH), F32),
        pltpu.VMEM((CH, DE), F32),
        pltpu.VMEM((CH, DE), F32),
        pltpu.VMEM((CH, 32), F32),
        pltpu.VMEM((CH, 32), F32),
        pltpu.VMEM((CH,), F32),
        pltpu.VMEM((CH,), F32),
        pltpu.VMEM((CH,), F32),
        pltpu.VMEM((CH,), F32),
        pltpu.VMEM((NPAD,), F32),
        pltpu.VMEM((NPAD,), F32),
        pltpu.VMEM((NPAD,), F32),
        pltpu.VMEM_SHARED((NPAD, DH), F32),
        pltpu.VMEM_SHARED((NPAD, 32), F32),
        pltpu.SemaphoreType.DMA,
        pltpu.SemaphoreType.DMA,
        pltpu.SemaphoreType.DMA,
        pltpu.SemaphoreType.DMA,
    ])
def _pass2(vfh, eah, srch2, dsth2, e1h, e2h, s1ph, s2ph,
           agg1ph, t2ph,
           dst_l, src_l, Vg0, Vg1, Msg0, Msg1, eag0, eag1, t2m0, t2m1,
           e1b0, e1b1, e2b0, e2b1, s1t, s2t, tmp, agg1, t2s,
           semG0, semG1, semS0, semS1):
    cid = lax.axis_index("c")
    sid = lax.axis_index("s")
    zero = jnp.zeros((L,), F32)
    iot = lax.iota(I32, L)
    BUFS = ((Vg0, Msg0, eag0, t2m0, e1b0, e2b0, semG0, semS0),
            (Vg1, Msg1, eag1, t2m1, e1b1, e2b1, semG1, semS1))

    def zcol(buf, ncols):
        def zb(i, _):
            def zc(j, _):
                jv = jnp.full((L,), 0, I32) + j
                plsc.store_scatter(buf, [iot + i * L, jv], zero)
                return 0
            lax.fori_loop(0, ncols, zc, 0)
            return 0
        lax.fori_loop(0, CH // L, zb, 0)
    zcol(Msg0, DH)
    zcol(t2m0, 32)
    for k in range(RPT // CH):
        base = sid * RPT + k * CH
        pltpu.sync_copy(Msg0, agg1.at[pl.ds(base, CH), :])
        pltpu.sync_copy(t2m0, t2s.at[pl.ds(base, CH), :])
    plsc.subcore_barrier()

    for ph, st in ((s1ph, s1t), (s2ph, s2t)):
        pltpu.sync_copy(ph.at[0], st)
        pltpu.sync_copy(ph.at[1], tmp)

        def sb(g, _):
            sl = pl.ds(g * L, L)
            st[sl] = st[sl] + tmp[sl]
            return 0
        lax.fori_loop(0, NPAD // L, sb, 0)

    offv = jnp.full((L,), 0, I32) + cid * N

    def fire_in(go, c, b):
        Vg, _, eag, _, e1b, e2b, semG, _ = BUFS[b]
        off = (go + c) * CH
        pltpu.async_copy(vfh.at[src_l.at[c]], Vg, semG)
        pltpu.async_copy(eah.at[pl.ds(off, CH), :], eag, semG)
        pltpu.async_copy(e1h.at[pl.ds(off, CH)], e1b, semG)
        pltpu.async_copy(e2h.at[pl.ds(off, CH)], e2b, semG)

    def wait_in(b):
        Vg, _, eag, _, e1b, e2b, semG, _ = BUFS[b]
        pltpu.make_async_copy(vfh.at[pl.ds(0, CH), :], Vg, semG).wait()
        pltpu.make_async_copy(eah.at[pl.ds(0, CH), :], eag, semG).wait()
        pltpu.make_async_copy(e1h.at[pl.ds(0, CH)], e1b, semG).wait()
        pltpu.make_async_copy(e2h.at[pl.ds(0, CH)], e2b, semG).wait()

    def fire_sc(c, b):
        _, Msg, _, t2m, _, _, _, semS = BUFS[b]
        pltpu.async_copy(Msg, agg1.at[dst_l.at[c]], semS, add=True)

        @pl.when(cid == 0)
        def _():
            pltpu.async_copy(t2m, t2s.at[dst_l.at[c]], semS, add=True)

    def drain_sc(b):
        _, Msg, _, t2m, _, _, _, semS = BUFS[b]
        pltpu.make_async_copy(vfh.at[pl.ds(0, CH), :], Msg, semS).wait()

        @pl.when(cid == 0)
        def _():
            pltpu.make_async_copy(t2ph.at[0, pl.ds(0, CH), :], t2m,
                                  semS).wait()

    rows_g = tuple(iot + g * L for g in range(NG))

    def compute(c, b):
        Vg, Msg, eag, t2m, e1b, e2b, _, _ = BUFS[b]
        w1s, w2s = [], []
        for g in range(NG):
            sl = pl.ds(g * L, L)
            dstv = dst_l[c, sl]
            s1g = plsc.load_gather(s1t, [dstv])
            s2g = plsc.load_gather(s2t, [dstv])
            w1s.append(e1b[sl] / (s1g + 1e-16))
            w2s.append(e2b[sl] / (s2g + 1e-16))

        def mc2(jj, _):
            for u in range(2):
                jv = jnp.full((L,), 0, I32) + (jj * 2 + u)
                for g in range(NG):
                    vc = plsc.load_gather(Vg, [rows_g[g], jv])
                    plsc.store_scatter(Msg, [rows_g[g], jv], vc * w1s[g])
            return 0
        lax.fori_loop(0, DH // 2, mc2, 0)

        @pl.when(cid == 0)
        def _():
            for j in range(DE):
                jv = jnp.full((L,), j, I32)
                for g in range(NG):
                    eac = plsc.load_gather(eag, [rows_g[g], jv])
                    plsc.store_scatter(t2m, [rows_g[g], jv], eac * w2s[g])
            jvq = jnp.full((L,), DE, I32)
            for g in range(NG):
                plsc.store_scatter(t2m, [rows_g[g], jvq], w2s[g])

    for h in range(NSEG):
        trow = sid * NCH2 + h * NCH2S
        pltpu.sync_copy(dsth2.at[pl.ds(trow, NCH2S), :], dst_l)
        pltpu.sync_copy(srch2.at[pl.ds(trow, NCH2S), :], src_l)

        def soff(r, _):
            for g in range(NG):
                sl = pl.ds(g * L, L)
                src_l[r, sl] = src_l[r, sl] + offv
            return 0
        lax.fori_loop(0, NCH2S, soff, 0)

        fire_in(trow, 0, 0)
        fire_in(trow, 1, 1)

        def pair(k, _):
            for b in (0, 1):
                c = k * 2 + b
                wait_in(b)

                @pl.when(k > 0)
                def _():
                    drain_sc(b)
                compute(c, b)
                fire_sc(c, b)
                cn = c + 2

                @pl.when(cn < NCH2S)
                def _():
                    fire_in(trow, cn, b)
            return 0
        lax.fori_loop(0, NCH2S // 2, pair, 0)
        drain_sc(0)
        drain_sc(1)

    plsc.subcore_barrier()
    pltpu.sync_copy(agg1.at[pl.ds(sid * RPT, RPT), :],
                    agg1ph.at[cid, pl.ds(sid * RPT, RPT), :])
    pltpu.sync_copy(t2s.at[pl.ds(sid * RPT, RPT), :],
                    t2ph.at[cid, pl.ds(sid * RPT, RPT), :])


def _fin_body(ag_ref, t2_ref, wh_ref, we3_ref, be3_ref, o_ref):
    a = jnp.concatenate([ag_ref[0], ag_ref[1]], axis=1)
    t2 = t2_ref[0] + t2_ref[1]
    wh = wh_ref[...]
    w23 = jnp.dot(we3_ref[...], wh, preferred_element_type=F32)
    bw = jnp.dot(be3_ref[...], wh, preferred_element_type=F32)
    o = (jnp.dot(a, wh, preferred_element_type=F32)
         + jnp.dot(t2[:, :DE], w23, preferred_element_type=F32)
         + t2[:, DE:DE + 1] * bw)
    o_ref[...] = o


def _finish(agg1p, t2p, Wh, We3, be3row):
    bm = 1000
    grid = (N // bm,)
    return pl.pallas_call(
        _fin_body,
        grid=grid,
        in_specs=[pl.BlockSpec((NC, bm, DH), lambda i: (0, i, 0)),
                  pl.BlockSpec((NC, bm, 32), lambda i: (0, i, 0)),
                  pl.BlockSpec((D, D), lambda i: (0, 0)),
                  pl.BlockSpec((DE, D), lambda i: (0, 0)),
                  pl.BlockSpec((1, D), lambda i: (0, 0))],
        out_specs=pl.BlockSpec((bm, D), lambda i: (i, 0)),
        out_shape=jax.ShapeDtypeStruct((N, D), F32),
    )(agg1p, t2p, Wh, We3, be3row)


def kernel(x, edge_index, edge_attr, Wq, bq, Wk, bk, Wv, bv,
           We2, be2, We3, be3, Wh):
    ei = edge_index.astype(I32)
    src2 = ei[0].reshape(E // CH, CH)
    dst2 = ei[1].reshape(E // CH, CH)
    W2p = jnp.concatenate(
        [We2.T, be2[:, None], jnp.zeros((D, 32 - DE - 1), F32)], axis=1)
    Q, K, V, q2s = _qkv(x, Wq, bq[None, :], Wk, bk[None, :],
                        Wv, bv[None, :], W2p)
    e1, e2, s1p, s2p = _pass1(Q, K, q2s, edge_attr, src2, dst2)
    vflat = jnp.concatenate([V[:, :DH], V[:, DH:]], axis=0)
    agg1p, t2p = _pass2(vflat, edge_attr, src2, dst2, e1, e2, s1p, s2p)
    return _finish(agg1p, t2p, Wh, We3, be3[None, :])


# speedup vs baseline: 1.0725x; 1.0725x over previous
import functools
import math

import jax
import jax.numpy as jnp
from jax import lax
from jax.experimental import pallas as pl
from jax.experimental.pallas import tpu as pltpu
from jax.experimental.pallas import tpu_sc as plsc

N = 10000
E = 320000
D = 128
DE = 16
NC, NS, L = 2, 16, 16
NW = NC * NS
CH = 80
NG = CH // L
EPT = E // NW
NCH1 = EPT // CH
EPS = E // NS
NCH2 = EPS // CH
DH = D // NC
NPAD = 10240
RPT = NPAD // NS
INV_SQRT_C = 1.0 / math.sqrt(float(D))
F32 = jnp.float32
I32 = jnp.int32

_mesh = plsc.VectorSubcoreMesh(
    core_axis_name="c", subcore_axis_name="s", num_cores=NC, num_subcores=NS)
_params = pltpu.CompilerParams(
    needs_layout_passes=False, use_tc_tiling_on_sc=False)


def _qkv_body(x_ref, wq_ref, bq_ref, wk_ref, bk_ref, wv_ref, bv_ref, w2p_ref,
              q_ref, k_ref, v_ref, q2_ref):
    xb = x_ref[...]
    q = jnp.dot(xb, wq_ref[...], preferred_element_type=F32) + bq_ref[...]
    k = jnp.dot(xb, wk_ref[...], preferred_element_type=F32) + bk_ref[...]
    v = jnp.dot(xb, wv_ref[...], preferred_element_type=F32) + bv_ref[...]
    q_ref[...] = q
    k_ref[...] = k
    v_ref[...] = v
    q2_ref[...] = jnp.dot(q, w2p_ref[...], preferred_element_type=F32)


def _qkv(x, Wq, bq, Wk, bk, Wv, bv, W2p):
    bm = 1000
    grid = (N // bm,)
    wspec = pl.BlockSpec((D, D), lambda i: (0, 0))
    bspec = pl.BlockSpec((1, D), lambda i: (0, 0))
    rspec = pl.BlockSpec((bm, D), lambda i: (i, 0))
    return pl.pallas_call(
        _qkv_body,
        grid=grid,
        in_specs=[rspec, wspec, bspec, wspec, bspec, wspec, bspec,
                  pl.BlockSpec((D, 32), lambda i: (0, 0))],
        out_specs=[rspec, rspec, rspec,
                   pl.BlockSpec((bm, 32), lambda i: (i, 0))],
        out_shape=[jax.ShapeDtypeStruct((N, D), F32),
                   jax.ShapeDtypeStruct((N, D), F32),
                   jax.ShapeDtypeStruct((N, D), F32),
                   jax.ShapeDtypeStruct((N, 32), F32)],
    )(x, Wq, bq, Wk, bk, Wv, bv, W2p)


@functools.partial(
    pl.kernel,
    out_type=(jax.ShapeDtypeStruct((E,), F32),
              jax.ShapeDtypeStruct((E,), F32),
              jax.ShapeDtypeStruct((NC, NPAD), F32),
              jax.ShapeDtypeStruct((NC, NPAD), F32)),
    mesh=_mesh,
    compiler_params=_params,
    scratch_types=[
        pltpu.VMEM((NCH1, CH), I32),
        pltpu.VMEM((NCH1, CH), I32),
        pltpu.VMEM((CH, D + 32), F32),
        pltpu.VMEM((CH, D + 32), F32),
        pltpu.VMEM((CH, D), F32),
        pltpu.VMEM((CH, D), F32),
        pltpu.VMEM((CH, DE), F32),
        pltpu.VMEM((CH, DE), F32),
        pltpu.VMEM((CH,), F32),
        pltpu.VMEM((CH,), F32),
        pltpu.VMEM((CH,), F32),
        pltpu.VMEM((CH,), F32),
        pltpu.VMEM((NPAD,), F32),
        pltpu.VMEM((NPAD,), F32),
        pltpu.VMEM((NS, RPT), F32),
        pltpu.VMEM((RPT,), F32),
        pltpu.VMEM_SHARED((NS, NPAD), F32),
        pltpu.VMEM_SHARED((NS, NPAD), F32),
        pltpu.SemaphoreType.DMA,
        pltpu.SemaphoreType.DMA,
        pltpu.SemaphoreType.DMA,
        pltpu.SemaphoreType.DMA,
    ])
def _pass1(qh, kh, eah, srch2, dsth2,
           e1h, e2h, s1ph, s2ph,
           dst_l, src_l, Qg0, Qg1, Kg0, Kg1, eag0, eag1,
           e1b0, e1b1, e2b0, e2b1, s1l, s2l, red, rowout, sh1, sh2,
           semG0, semG1, semW0, semW1):
    cid = lax.axis_index("c")
    sid = lax.axis_index("s")
    wid = cid * NS + sid
    zero = jnp.zeros((L,), F32)
    iot = lax.iota(I32, L)
    BUFS = ((Qg0, Kg0, eag0, e1b0, e2b0, semG0, semW0),
            (Qg1, Kg1, eag1, e1b1, e2b1, semG1, semW1))

    def zbody(i, _):
        s1l[pl.ds(i * L, L)] = zero
        s2l[pl.ds(i * L, L)] = zero
        return 0
    lax.fori_loop(0, NPAD // L, zbody, 0)

    pltpu.sync_copy(dsth2.at[pl.ds(wid * NCH1, NCH1), :], dst_l)
    pltpu.sync_copy(srch2.at[pl.ds(wid * NCH1, NCH1), :], src_l)

    def fire_inputs(c, b):
        Qg, Kg, eag, _, _, semG, _ = BUFS[b]
        off = (wid * NCH1 + c) * CH
        pltpu.async_copy(qh.at[dst_l.at[c]], Qg, semG)
        pltpu.async_copy(kh.at[src_l.at[c]], Kg, semG)
        pltpu.async_copy(eah.at[pl.ds(off, CH), :], eag, semG)

    def wait_inputs(b):
        Qg, Kg, eag, _, _, semG, _ = BUFS[b]
        pltpu.make_async_copy(qh.at[pl.ds(0, CH), :], Qg, semG).wait()
        pltpu.make_async_copy(kh.at[pl.ds(0, CH), :], Kg, semG).wait()
        pltpu.make_async_copy(eah.at[pl.ds(0, CH), :], eag, semG).wait()

    def fire_writes(c, b):
        _, _, _, e1b, e2b, _, semW = BUFS[b]
        off = (wid * NCH1 + c) * CH
        pltpu.async_copy(e1b, e1h.at[pl.ds(off, CH)], semW)
        pltpu.async_copy(e2b, e2h.at[pl.ds(off, CH)], semW)

    def drain_writes(b):
        _, _, _, e1b, e2b, _, semW = BUFS[b]
        pltpu.make_async_copy(e1h.at[pl.ds(0, CH)], e1b, semW).wait()
        pltpu.make_async_copy(e2h.at[pl.ds(0, CH)], e2b, semW).wait()

    def compute(c, b):
        Qg, Kg, eag, e1b, e2b, _, _ = BUFS[b]

        def group(gi, _):
            rows = iot + gi * L

            def dot4(jj, acc):
                for u in range(4):
                    jv = jnp.full((L,), 0, I32) + (jj * 4 + u)
                    acc = acc + (plsc.load_gather(Qg, [rows, jv])
                                 * plsc.load_gather(Kg, [rows, jv]))
                return acc
            a1 = lax.fori_loop(0, D // 4, dot4, zero) * INV_SQRT_C

            a2acc = zero
            for j in range(DE):
                jv = jnp.full((L,), D + j, I32)
                jve = jnp.full((L,), j, I32)
                a2acc = a2acc + (plsc.load_gather(Qg, [rows, jv])
                                 * plsc.load_gather(eag, [rows, jve]))
            qb = plsc.load_gather(Qg, [rows, jnp.full((L,), D + DE, I32)])
            a2 = (a2acc + qb) * INV_SQRT_C

            e1 = jnp.exp(a1)
            e2 = jnp.exp(a2)
            e1b[pl.ds(gi * L, L)] = e1
            e2b[pl.ds(gi * L, L)] = e2
            dstv = dst_l[c, pl.ds(gi * L, L)]
            plsc.addupdate_scatter(s1l, [dstv], e1)
            plsc.addupdate_scatter(s2l, [dstv], e2)
            return 0
        lax.fori_loop(0, NG, group, 0)

    fire_inputs(0, 0)
    fire_inputs(1, 1)

    def pair(k, _):
        for b in (0, 1):
            c = k * 2 + b
            wait_inputs(b)

            @pl.when(k > 0)
            def _():
                drain_writes(b)
            compute(c, b)
            fire_writes(c, b)
            cn = c + 2

            @pl.when(cn < NCH1)
            def _():
                fire_inputs(cn, b)
        return 0
    lax.fori_loop(0, NCH1 // 2, pair, 0)
    wait_inputs(0)
    drain_writes(0)
    compute(NCH1 - 1, 0)
    fire_writes(NCH1 - 1, 0)
    drain_writes(1)
    drain_writes(0)

    pltpu.sync_copy(s1l, sh1.at[sid])
    pltpu.sync_copy(s2l, sh2.at[sid])
    plsc.subcore_barrier()
    for sh, outref in ((sh1, s1ph), (sh2, s2ph)):
        pltpu.sync_copy(sh.at[:, pl.ds(sid * RPT, RPT)], red)

        def rbody(g, _):
            acc = red[0, pl.ds(g * L, L)]
            for r in range(1, NS):
                acc = acc + red[r, pl.ds(g * L, L)]
            rowout[pl.ds(g * L, L)] = acc
            return 0
        lax.fori_loop(0, RPT // L, rbody, 0)
        pltpu.sync_copy(rowout, outref.at[cid, pl.ds(sid * RPT, RPT)])


NSEG = 5
NCH2S = NCH2 // NSEG


@functools.partial(
    pl.kernel,
    out_type=(jax.ShapeDtypeStruct((NC, NPAD, DH), F32),
              jax.ShapeDtypeStruct((NC, NPAD, 32), F32)),
    mesh=_mesh,
    compiler_params=_params,
    scratch_types=[
        pltpu.VMEM((NCH2S, CH), I32),
        pltpu.VMEM((NCH2S, CH), I32),
        pltpu.VMEM((CH, DH), F32),
        pltpu.VMEM((CH, DH), F32),
        pltpu.VMEM((CH, DH), F32),
        pltpu.VMEM((CH, DH), F32),
        pltpu.VMEM((CH, DE), F32),
        pltpu.VMEM((CH, DE), F32),
        pltpu.VMEM((CH, 32), F32),
        pltpu.VMEM((CH, 32), F32),
        pltpu.VMEM((CH,), F32),
        pltpu.VMEM((CH,), F32),
        pltpu.VMEM((CH,), F32),
        pltpu.VMEM((CH,), F32),
        pltpu.VMEM((NPAD,), F32),
        pltpu.VMEM((NPAD,), F32),
        pltpu.VMEM((NPAD,), F32),
        pltpu.VMEM_SHARED((NPAD, DH), F32),
        pltpu.VMEM_SHARED((NPAD, 32), F32),
        pltpu.SemaphoreType.DMA,
        pltpu.SemaphoreType.DMA,
        pltpu.SemaphoreType.DMA,
        pltpu.SemaphoreType.DMA,
    ])
def _pass2(vfh, eah, srch2, dsth2, e1h, e2h, s1ph, s2ph,
           agg1ph, t2ph,
           dst_l, src_l, Vg0, Vg1, Msg0, Msg1, eag0, eag1, t2m0, t2m1,
           e1b0, e1b1, e2b0, e2b1, s1t, s2t, tmp, agg1, t2s,
           semG0, semG1, semS0, semS1):
    cid = lax.axis_index("c")
    sid = lax.axis_index("s")
    zero = jnp.zeros((L,), F32)
    iot = lax.iota(I32, L)
    BUFS = ((Vg0, Msg0, eag0, t2m0, e1b0, e2b0, semG0, semS0),
            (Vg1, Msg1, eag1, t2m1, e1b1, e2b1, semG1, semS1))

    def zcol(buf, ncols):
        def zb(i, _):
            def zc(j, _):
                jv = jnp.full((L,), 0, I32) + j
                plsc.store_scatter(buf, [iot + i * L, jv], zero)
                return 0
            lax.fori_loop(0, ncols, zc, 0)
            return 0
        lax.fori_loop(0, CH // L, zb, 0)
    zcol(Msg0, DH)
    zcol(t2m0, 32)
    for k in range(RPT // CH):
        base = sid * RPT + k * CH
        pltpu.sync_copy(Msg0, agg1.at[pl.ds(base, CH), :])
        pltpu.sync_copy(t2m0, t2s.at[pl.ds(base, CH), :])
    plsc.subcore_barrier()

    for ph, st in ((s1ph, s1t), (s2ph, s2t)):
        pltpu.sync_copy(ph.at[0], st)
        pltpu.sync_copy(ph.at[1], tmp)

        def sb(g, _):
            sl = pl.ds(g * L, L)
            st[sl] = st[sl] + tmp[sl]
            return 0
        lax.fori_loop(0, NPAD // L, sb, 0)

    offv = jnp.full((L,), 0, I32) + cid * N

    def fire_in(go, c, b):
        Vg, _, eag, _, e1b, e2b, semG, _ = BUFS[b]
        off = (go + c) * CH
        pltpu.async_copy(vfh.at[src_l.at[c]], Vg, semG)
        pltpu.async_copy(eah.at[pl.ds(off, CH), :], eag, semG)
        pltpu.async_copy(e1h.at[pl.ds(off, CH)], e1b, semG)
        pltpu.async_copy(e2h.at[pl.ds(off, CH)], e2b, semG)

    def wait_in(b):
        Vg, _, eag, _, e1b, e2b, semG, _ = BUFS[b]
        pltpu.make_async_copy(vfh.at[pl.ds(0, CH), :], Vg, semG).wait()
        pltpu.make_async_copy(eah.at[pl.ds(0, CH), :], eag, semG).wait()
        pltpu.make_async_copy(e1h.at[pl.ds(0, CH)], e1b, semG).wait()
        pltpu.make_async_copy(e2h.at[pl.ds(0, CH)], e2b, semG).wait()

    def fire_sc(c, b):
        _, Msg, _, t2m, _, _, _, semS = BUFS[b]
        pltpu.async_copy(Msg, agg1.at[dst_l.at[c]], semS, add=True)

        @pl.when(cid == 0)
        def _():
            pltpu.async_copy(t2m, t2s.at[dst_l.at[c]], semS, add=True)

    def drain_sc(b):
        _, Msg, _, t2m, _, _, _, semS = BUFS[b]
        pltpu.make_async_copy(vfh.at[pl.ds(0, CH), :], Msg, semS).wait()

        @pl.when(cid == 0)
        def _():
            pltpu.make_async_copy(t2ph.at[0, pl.ds(0, CH), :], t2m,
                                  semS).wait()

    rows_g = tuple(iot + g * L for g in range(NG))

    def compute(c, b):
        Vg, Msg, eag, t2m, e1b, e2b, _, _ = BUFS[b]
        w1s, w2s = [], []
        for g in range(NG):
            sl = pl.ds(g * L, L)
            dstv = dst_l[c, sl]
            s1g = plsc.load_gather(s1t, [dstv])
            s2g = plsc.load_gather(s2t, [dstv])
            w1s.append(e1b[sl] / (s1g + 1e-16))
            w2s.append(e2b[sl] / (s2g + 1e-16))

        def mc2(jj, _):
            for u in range(2):
                jv = jnp.full((L,), 0, I32) + (jj * 2 + u)
                for g in range(NG):
                    vc = plsc.load_gather(Vg, [rows_g[g], jv])
                    plsc.store_scatter(Msg, [rows_g[g], jv], vc * w1s[g])
            return 0
        lax.fori_loop(0, DH // 2, mc2, 0)

        @pl.when(cid == 0)
        def _():
            for j in range(DE):
                jv = jnp.full((L,), j, I32)
                for g in range(NG):
                    eac = plsc.load_gather(eag, [rows_g[g], jv])
                    plsc.store_scatter(t2m, [rows_g[g], jv], eac * w2s[g])
            jvq = jnp.full((L,), DE, I32)
            for g in range(NG):
                plsc.store_scatter(t2m, [rows_g[g], jvq], w2s[g])

    for h in range(NSEG):
        trow = sid * NCH2 + h * NCH2S
        pltpu.sync_copy(dsth2.at[pl.ds(trow, NCH2S), :], dst_l)
        pltpu.sync_copy(srch2.at[pl.ds(trow, NCH2S), :], src_l)

        def soff(r, _):
            for g in range(NG):
                sl = pl.ds(g * L, L)
                src_l[r, sl] = src_l[r, sl] + offv
            return 0
        lax.fori_loop(0, NCH2S, soff, 0)

        fire_in(trow, 0, 0)
        fire_in(trow, 1, 1)

        def pair(k, _):
            for b in (0, 1):
                c = k * 2 + b
                wait_in(b)

                @pl.when(k > 0)
                def _():
                    drain_sc(b)
                compute(c, b)
                fire_sc(c, b)
                cn = c + 2

                @pl.when(cn < NCH2S)
                def _():
                    fire_in(trow, cn, b)
            return 0
        lax.fori_loop(0, NCH2S // 2, pair, 0)
        drain_sc(0)
        drain_sc(1)

    plsc.subcore_barrier()
    pltpu.sync_copy(agg1.at[pl.ds(sid * RPT, RPT), :],
                    agg1ph.at[cid, pl.ds(sid * RPT, RPT), :])
    pltpu.sync_copy(t2s.at[pl.ds(sid * RPT, RPT), :],
                    t2ph.at[cid, pl.ds(sid * RPT, RPT), :])


def _fin_body(ag_ref, t2_ref, wh_ref, we3_ref, be3_ref, o_ref):
    a = jnp.concatenate([ag_ref[0], ag_ref[1]], axis=1)
    t2 = t2_ref[0] + t2_ref[1]
    wh = wh_ref[...]
    w23 = jnp.dot(we3_ref[...], wh, preferred_element_type=F32)
    bw = jnp.dot(be3_ref[...], wh, preferred_element_type=F32)
    o = (jnp.dot(a, wh, preferred_element_type=F32)
         + jnp.dot(t2[:, :DE], w23, preferred_element_type=F32)
         + t2[:, DE:DE + 1] * bw)
    o_ref[...] = o


def _finish(agg1p, t2p, Wh, We3, be3row):
    bm = 1000
    grid = (N // bm,)
    return pl.pallas_call(
        _fin_body,
        grid=grid,
        in_specs=[pl.BlockSpec((NC, bm, DH), lambda i: (0, i, 0)),
                  pl.BlockSpec((NC, bm, 32), lambda i: (0, i, 0)),
                  pl.BlockSpec((D, D), lambda i: (0, 0)),
                  pl.BlockSpec((DE, D), lambda i: (0, 0)),
                  pl.BlockSpec((1, D), lambda i: (0, 0))],
        out_specs=pl.BlockSpec((bm, D), lambda i: (i, 0)),
        out_shape=jax.ShapeDtypeStruct((N, D), F32),
    )(agg1p, t2p, Wh, We3, be3row)


def kernel(x, edge_index, edge_attr, Wq, bq, Wk, bk, Wv, bv,
           We2, be2, We3, be3, Wh):
    ei = edge_index.astype(I32)
    src2 = ei[0].reshape(E // CH, CH)
    dst2 = ei[1].reshape(E // CH, CH)
    W2p = jnp.concatenate(
        [We2.T, be2[:, None], jnp.zeros((D, 32 - DE - 1), F32)], axis=1)
    Q, K, V, q2s = _qkv(x, Wq, bq[None, :], Wk, bk[None, :],
                        Wv, bv[None, :], W2p)
    qq2 = jnp.concatenate([Q, q2s], axis=1)
    e1, e2, s1p, s2p = _pass1(qq2, K, edge_attr, src2, dst2)
    vflat = jnp.concatenate([V[:, :DH], V[:, DH:]], axis=0)
    agg1p, t2p = _pass2(vflat, edge_attr, src2, dst2, e1, e2, s1p, s2p)
    return _finish(agg1p, t2p, Wh, We3, be3[None, :])


# speedup vs baseline: 1.5420x; 1.4378x over previous
import functools
import math

import jax
import jax.numpy as jnp
from jax import lax
from jax.experimental import pallas as pl
from jax.experimental.pallas import tpu as pltpu
from jax.experimental.pallas import tpu_sc as plsc

N = 10000
E = 320000
D = 128
DE = 16
NC, NS, L = 2, 16, 16
NW = NC * NS
CH = 80
NG = CH // L
EPT = E // NW
NCH1 = EPT // CH
EPS = E // NS
NCH2 = EPS // CH
DH = D // NC
NPAD = 10240
RPT = NPAD // NS
INV_SQRT_C = 1.0 / math.sqrt(float(D))
F32 = jnp.float32
I32 = jnp.int32

_mesh = plsc.VectorSubcoreMesh(
    core_axis_name="c", subcore_axis_name="s", num_cores=NC, num_subcores=NS)
_params = pltpu.CompilerParams(
    needs_layout_passes=False, use_tc_tiling_on_sc=False)


def _qkv_body(x_ref, wq_ref, bq_ref, wk_ref, bk_ref, wv_ref, bv_ref, w2p_ref,
              q_ref, k_ref, v_ref, q2_ref):
    xb = x_ref[...]
    q = jnp.dot(xb, wq_ref[...], preferred_element_type=F32) + bq_ref[...]
    k = jnp.dot(xb, wk_ref[...], preferred_element_type=F32) + bk_ref[...]
    v = jnp.dot(xb, wv_ref[...], preferred_element_type=F32) + bv_ref[...]
    q_ref[...] = q
    k_ref[...] = k
    v_ref[...] = v
    q2_ref[...] = jnp.dot(q, w2p_ref[...], preferred_element_type=F32)


def _qkv(x, Wq, bq, Wk, bk, Wv, bv, W2p):
    bm = 1000
    grid = (N // bm,)
    wspec = pl.BlockSpec((D, D), lambda i: (0, 0))
    bspec = pl.BlockSpec((1, D), lambda i: (0, 0))
    rspec = pl.BlockSpec((bm, D), lambda i: (i, 0))
    return pl.pallas_call(
        _qkv_body,
        grid=grid,
        in_specs=[rspec, wspec, bspec, wspec, bspec, wspec, bspec,
                  pl.BlockSpec((D, 32), lambda i: (0, 0))],
        out_specs=[rspec, rspec, rspec,
                   pl.BlockSpec((bm, 32), lambda i: (i, 0))],
        out_shape=[jax.ShapeDtypeStruct((N, D), F32),
                   jax.ShapeDtypeStruct((N, D), F32),
                   jax.ShapeDtypeStruct((N, D), F32),
                   jax.ShapeDtypeStruct((N, 32), F32)],
    )(x, Wq, bq, Wk, bk, Wv, bv, W2p)


@functools.partial(
    pl.kernel,
    out_type=(jax.ShapeDtypeStruct((E,), F32),
              jax.ShapeDtypeStruct((E,), F32),
              jax.ShapeDtypeStruct((NC, NPAD), F32),
              jax.ShapeDtypeStruct((NC, NPAD), F32)),
    mesh=_mesh,
    compiler_params=_params,
    scratch_types=[
        pltpu.VMEM((NCH1, CH), I32),
        pltpu.VMEM((NCH1, CH), I32),
        pltpu.VMEM((CH, D + 32), F32),
        pltpu.VMEM((CH, D + 32), F32),
        pltpu.VMEM((CH, D), F32),
        pltpu.VMEM((CH, D), F32),
        pltpu.VMEM((CH, DE), F32),
        pltpu.VMEM((CH, DE), F32),
        pltpu.VMEM((CH,), F32),
        pltpu.VMEM((CH,), F32),
        pltpu.VMEM((CH,), F32),
        pltpu.VMEM((CH,), F32),
        pltpu.VMEM((NPAD,), F32),
        pltpu.VMEM((NPAD,), F32),
        pltpu.VMEM((NS, RPT), F32),
        pltpu.VMEM((RPT,), F32),
        pltpu.VMEM_SHARED((NS, NPAD), F32),
        pltpu.VMEM_SHARED((NS, NPAD), F32),
        pltpu.SemaphoreType.DMA,
        pltpu.SemaphoreType.DMA,
        pltpu.SemaphoreType.DMA,
        pltpu.SemaphoreType.DMA,
    ])
def _pass1(qh, kh, eah, srch2, dsth2,
           e1h, e2h, s1ph, s2ph,
           dst_l, src_l, Qg0, Qg1, Kg0, Kg1, eag0, eag1,
           e1b0, e1b1, e2b0, e2b1, s1l, s2l, red, rowout, sh1, sh2,
           semG0, semG1, semW0, semW1):
    cid = lax.axis_index("c")
    sid = lax.axis_index("s")
    wid = cid * NS + sid
    zero = jnp.zeros((L,), F32)
    iot = lax.iota(I32, L)
    BUFS = ((Qg0, Kg0, eag0, e1b0, e2b0, semG0, semW0),
            (Qg1, Kg1, eag1, e1b1, e2b1, semG1, semW1))

    def zbody(i, _):
        s1l[pl.ds(i * L, L)] = zero
        s2l[pl.ds(i * L, L)] = zero
        return 0
    lax.fori_loop(0, NPAD // L, zbody, 0)

    pltpu.sync_copy(dsth2.at[pl.ds(wid * NCH1, NCH1), :], dst_l)
    pltpu.sync_copy(srch2.at[pl.ds(wid * NCH1, NCH1), :], src_l)

    def fire_inputs(c, b):
        Qg, Kg, eag, _, _, semG, _ = BUFS[b]
        off = (wid * NCH1 + c) * CH
        pltpu.async_copy(qh.at[dst_l.at[c]], Qg, semG)
        pltpu.async_copy(kh.at[src_l.at[c]], Kg, semG)
        pltpu.async_copy(eah.at[pl.ds(off, CH), :], eag, semG)

    def wait_inputs(b):
        Qg, Kg, eag, _, _, semG, _ = BUFS[b]
        pltpu.make_async_copy(qh.at[pl.ds(0, CH), :], Qg, semG).wait()
        pltpu.make_async_copy(kh.at[pl.ds(0, CH), :], Kg, semG).wait()
        pltpu.make_async_copy(eah.at[pl.ds(0, CH), :], eag, semG).wait()

    def fire_writes(c, b):
        _, _, _, e1b, e2b, _, semW = BUFS[b]
        off = (wid * NCH1 + c) * CH
        pltpu.async_copy(e1b, e1h.at[pl.ds(off, CH)], semW)
        pltpu.async_copy(e2b, e2h.at[pl.ds(off, CH)], semW)

    def drain_writes(b):
        _, _, _, e1b, e2b, _, semW = BUFS[b]
        pltpu.make_async_copy(e1h.at[pl.ds(0, CH)], e1b, semW).wait()
        pltpu.make_async_copy(e2h.at[pl.ds(0, CH)], e2b, semW).wait()

    def compute(c, b):
        Qg, Kg, eag, e1b, e2b, _, _ = BUFS[b]

        def group(gi, _):
            rows = iot + gi * L

            def dot4(j, accs):
                out = []
                for u in range(4):
                    jv = jnp.full((L,), 0, I32) + (j + u)
                    out.append(accs[u] + (plsc.load_gather(Qg, [rows, jv])
                                          * plsc.load_gather(Kg, [rows, jv])))
                return tuple(out)
            accs = plsc.parallel_loop(0, D, 4, unroll=4,
                                      carry=(zero, zero, zero, zero))(dot4)
            a1 = ((accs[0] + accs[1]) + (accs[2] + accs[3])) * INV_SQRT_C

            a2acc = zero
            for j in range(DE):
                jv = jnp.full((L,), D + j, I32)
                jve = jnp.full((L,), j, I32)
                a2acc = a2acc + (plsc.load_gather(Qg, [rows, jv])
                                 * plsc.load_gather(eag, [rows, jve]))
            qb = plsc.load_gather(Qg, [rows, jnp.full((L,), D + DE, I32)])
            a2 = (a2acc + qb) * INV_SQRT_C

            e1 = jnp.exp(a1)
            e2 = jnp.exp(a2)
            e1b[pl.ds(gi * L, L)] = e1
            e2b[pl.ds(gi * L, L)] = e2
            dstv = dst_l[c, pl.ds(gi * L, L)]
            plsc.addupdate_scatter(s1l, [dstv], e1)
            plsc.addupdate_scatter(s2l, [dstv], e2)
            return 0
        lax.fori_loop(0, NG, group, 0)

    fire_inputs(0, 0)
    fire_inputs(1, 1)

    def pair(k, _):
        for b in (0, 1):
            c = k * 2 + b
            wait_inputs(b)

            @pl.when(k > 0)
            def _():
                drain_writes(b)
            compute(c, b)
            fire_writes(c, b)
            cn = c + 2

            @pl.when(cn < NCH1)
            def _():
                fire_inputs(cn, b)
        return 0
    lax.fori_loop(0, NCH1 // 2, pair, 0)
    wait_inputs(0)
    drain_writes(0)
    compute(NCH1 - 1, 0)
    fire_writes(NCH1 - 1, 0)
    drain_writes(1)
    drain_writes(0)

    pltpu.sync_copy(s1l, sh1.at[sid])
    pltpu.sync_copy(s2l, sh2.at[sid])
    plsc.subcore_barrier()
    for sh, outref in ((sh1, s1ph), (sh2, s2ph)):
        pltpu.sync_copy(sh.at[:, pl.ds(sid * RPT, RPT)], red)

        def rbody(g, _):
            acc = red[0, pl.ds(g * L, L)]
            for r in range(1, NS):
                acc = acc + red[r, pl.ds(g * L, L)]
            rowout[pl.ds(g * L, L)] = acc
            return 0
        lax.fori_loop(0, RPT // L, rbody, 0)
        pltpu.sync_copy(rowout, outref.at[cid, pl.ds(sid * RPT, RPT)])


NSEG = 5
NCH2S = NCH2 // NSEG


@functools.partial(
    pl.kernel,
    out_type=(jax.ShapeDtypeStruct((NC, NPAD, DH), F32),
              jax.ShapeDtypeStruct((NC, NPAD, 32), F32)),
    mesh=_mesh,
    compiler_params=_params,
    scratch_types=[
        pltpu.VMEM((NCH2S, CH), I32),
        pltpu.VMEM((NCH2S, CH), I32),
        pltpu.VMEM((CH, DH), F32),
        pltpu.VMEM((CH, DH), F32),
        pltpu.VMEM((CH, DH), F32),
        pltpu.VMEM((CH, DH), F32),
        pltpu.VMEM((CH, DE), F32),
        pltpu.VMEM((CH, DE), F32),
        pltpu.VMEM((CH, 32), F32),
        pltpu.VMEM((CH, 32), F32),
        pltpu.VMEM((CH,), F32),
        pltpu.VMEM((CH,), F32),
        pltpu.VMEM((CH,), F32),
        pltpu.VMEM((CH,), F32),
        pltpu.VMEM((NPAD,), F32),
        pltpu.VMEM((NPAD,), F32),
        pltpu.VMEM((NPAD,), F32),
        pltpu.VMEM_SHARED((NPAD, DH), F32),
        pltpu.VMEM_SHARED((NPAD, 32), F32),
        pltpu.SemaphoreType.DMA,
        pltpu.SemaphoreType.DMA,
        pltpu.SemaphoreType.DMA,
        pltpu.SemaphoreType.DMA,
    ])
def _pass2(vfh, eah, srch2, dsth2, e1h, e2h, s1ph, s2ph,
           agg1ph, t2ph,
           dst_l, src_l, Vg0, Vg1, Msg0, Msg1, eag0, eag1, t2m0, t2m1,
           e1b0, e1b1, e2b0, e2b1, s1t, s2t, tmp, agg1, t2s,
           semG0, semG1, semS0, semS1):
    cid = lax.axis_index("c")
    sid = lax.axis_index("s")
    zero = jnp.zeros((L,), F32)
    iot = lax.iota(I32, L)
    BUFS = ((Vg0, Msg0, eag0, t2m0, e1b0, e2b0, semG0, semS0),
            (Vg1, Msg1, eag1, t2m1, e1b1, e2b1, semG1, semS1))

    def zcol(buf, ncols):
        def zb(i, _):
            def zc(j, _):
                jv = jnp.full((L,), 0, I32) + j
                plsc.store_scatter(buf, [iot + i * L, jv], zero)
                return 0
            lax.fori_loop(0, ncols, zc, 0)
            return 0
        lax.fori_loop(0, CH // L, zb, 0)
    zcol(Msg0, DH)
    zcol(t2m0, 32)
    for k in range(RPT // CH):
        base = sid * RPT + k * CH
        pltpu.sync_copy(Msg0, agg1.at[pl.ds(base, CH), :])
        pltpu.sync_copy(t2m0, t2s.at[pl.ds(base, CH), :])
    plsc.subcore_barrier()

    for ph, st in ((s1ph, s1t), (s2ph, s2t)):
        pltpu.sync_copy(ph.at[0], st)
        pltpu.sync_copy(ph.at[1], tmp)

        def sb(g, _):
            sl = pl.ds(g * L, L)
            st[sl] = st[sl] + tmp[sl]
            return 0
        lax.fori_loop(0, NPAD // L, sb, 0)

    offv = jnp.full((L,), 0, I32) + cid * N

    def fire_in(go, c, b):
        Vg, _, eag, _, e1b, e2b, semG, _ = BUFS[b]
        off = (go + c) * CH
        pltpu.async_copy(vfh.at[src_l.at[c]], Vg, semG)
        pltpu.async_copy(eah.at[pl.ds(off, CH), :], eag, semG)
        pltpu.async_copy(e1h.at[pl.ds(off, CH)], e1b, semG)
        pltpu.async_copy(e2h.at[pl.ds(off, CH)], e2b, semG)

    def wait_in(b):
        Vg, _, eag, _, e1b, e2b, semG, _ = BUFS[b]
        pltpu.make_async_copy(vfh.at[pl.ds(0, CH), :], Vg, semG).wait()
        pltpu.make_async_copy(eah.at[pl.ds(0, CH), :], eag, semG).wait()
        pltpu.make_async_copy(e1h.at[pl.ds(0, CH)], e1b, semG).wait()
        pltpu.make_async_copy(e2h.at[pl.ds(0, CH)], e2b, semG).wait()

    def fire_sc(c, b):
        _, Msg, _, t2m, _, _, _, semS = BUFS[b]
        pltpu.async_copy(Msg, agg1.at[dst_l.at[c]], semS, add=True)

        @pl.when(cid == 0)
        def _():
            pltpu.async_copy(t2m, t2s.at[dst_l.at[c]], semS, add=True)

    def drain_sc(b):
        _, Msg, _, t2m, _, _, _, semS = BUFS[b]
        pltpu.make_async_copy(vfh.at[pl.ds(0, CH), :], Msg, semS).wait()

        @pl.when(cid == 0)
        def _():
            pltpu.make_async_copy(t2ph.at[0, pl.ds(0, CH), :], t2m,
                                  semS).wait()

    rows_g = tuple(iot + g * L for g in range(NG))

    def compute(c, b):
        Vg, Msg, eag, t2m, e1b, e2b, _, _ = BUFS[b]
        w1s, w2s = [], []
        for g in range(NG):
            sl = pl.ds(g * L, L)
            dstv = dst_l[c, sl]
            s1g = plsc.load_gather(s1t, [dstv])
            s2g = plsc.load_gather(s2t, [dstv])
            w1s.append(e1b[sl] / (s1g + 1e-16))
            w2s.append(e2b[sl] / (s2g + 1e-16))

        def mcol(j):
            jv = jnp.full((L,), 0, I32) + j
            for g in range(NG):
                vc = plsc.load_gather(Vg, [rows_g[g], jv])
                plsc.store_scatter(Msg, [rows_g[g], jv], vc * w1s[g])
        plsc.parallel_loop(0, DH, 1, unroll=4)(mcol)

        @pl.when(cid == 0)
        def _():
            for j in range(DE):
                jv = jnp.full((L,), j, I32)
                for g in range(NG):
                    eac = plsc.load_gather(eag, [rows_g[g], jv])
                    plsc.store_scatter(t2m, [rows_g[g], jv], eac * w2s[g])
            jvq = jnp.full((L,), DE, I32)
            for g in range(NG):
                plsc.store_scatter(t2m, [rows_g[g], jvq], w2s[g])

    for h in range(NSEG):
        trow = sid * NCH2 + h * NCH2S
        pltpu.sync_copy(dsth2.at[pl.ds(trow, NCH2S), :], dst_l)
        pltpu.sync_copy(srch2.at[pl.ds(trow, NCH2S), :], src_l)

        def soff(r, _):
            for g in range(NG):
                sl = pl.ds(g * L, L)
                src_l[r, sl] = src_l[r, sl] + offv
            return 0
        lax.fori_loop(0, NCH2S, soff, 0)

        fire_in(trow, 0, 0)
        fire_in(trow, 1, 1)

        def pair(k, _):
            for b in (0, 1):
                c = k * 2 + b
                wait_in(b)

                @pl.when(k > 0)
                def _():
                    drain_sc(b)
                compute(c, b)
                fire_sc(c, b)
                cn = c + 2

                @pl.when(cn < NCH2S)
                def _():
                    fire_in(trow, cn, b)
            return 0
        lax.fori_loop(0, NCH2S // 2, pair, 0)
        drain_sc(0)
        drain_sc(1)

    plsc.subcore_barrier()
    pltpu.sync_copy(agg1.at[pl.ds(sid * RPT, RPT), :],
                    agg1ph.at[cid, pl.ds(sid * RPT, RPT), :])
    pltpu.sync_copy(t2s.at[pl.ds(sid * RPT, RPT), :],
                    t2ph.at[cid, pl.ds(sid * RPT, RPT), :])


def _fin_body(ag_ref, t2_ref, wh_ref, we3_ref, be3_ref, o_ref):
    a = jnp.concatenate([ag_ref[0], ag_ref[1]], axis=1)
    t2 = t2_ref[0] + t2_ref[1]
    wh = wh_ref[...]
    w23 = jnp.dot(we3_ref[...], wh, preferred_element_type=F32)
    bw = jnp.dot(be3_ref[...], wh, preferred_element_type=F32)
    o = (jnp.dot(a, wh, preferred_element_type=F32)
         + jnp.dot(t2[:, :DE], w23, preferred_element_type=F32)
         + t2[:, DE:DE + 1] * bw)
    o_ref[...] = o


def _finish(agg1p, t2p, Wh, We3, be3row):
    bm = 1000
    grid = (N // bm,)
    return pl.pallas_call(
        _fin_body,
        grid=grid,
        in_specs=[pl.BlockSpec((NC, bm, DH), lambda i: (0, i, 0)),
                  pl.BlockSpec((NC, bm, 32), lambda i: (0, i, 0)),
                  pl.BlockSpec((D, D), lambda i: (0, 0)),
                  pl.BlockSpec((DE, D), lambda i: (0, 0)),
                  pl.BlockSpec((1, D), lambda i: (0, 0))],
        out_specs=pl.BlockSpec((bm, D), lambda i: (i, 0)),
        out_shape=jax.ShapeDtypeStruct((N, D), F32),
    )(agg1p, t2p, Wh, We3, be3row)


def kernel(x, edge_index, edge_attr, Wq, bq, Wk, bk, Wv, bv,
           We2, be2, We3, be3, Wh):
    ei = edge_index.astype(I32)
    src2 = ei[0].reshape(E // CH, CH)
    dst2 = ei[1].reshape(E // CH, CH)
    W2p = jnp.concatenate(
        [We2.T, be2[:, None], jnp.zeros((D, 32 - DE - 1), F32)], axis=1)
    Q, K, V, q2s = _qkv(x, Wq, bq[None, :], Wk, bk[None, :],
                        Wv, bv[None, :], W2p)
    qq2 = jnp.concatenate([Q, q2s], axis=1)
    e1, e2, s1p, s2p = _pass1(qq2, K, edge_attr, src2, dst2)
    vflat = jnp.concatenate([V[:, :DH], V[:, DH:]], axis=0)
    agg1p, t2p = _pass2(vflat, edge_attr, src2, dst2, e1, e2, s1p, s2p)
    return _finish(agg1p, t2p, Wh, We3, be3[None, :])
